# Initial kernel scaffold; baseline (speedup 1.0000x reference)
import jax
import jax.numpy as jnp
from jax.experimental import pallas as pl


def kernel(users, items, edge_index, user_emb, item_emb):
    raise NotImplementedError("write your pallas kernel here")


# speedup vs baseline: 6.1118x; 6.1118x over previous
import functools

import jax
import jax.numpy as jnp
from jax import lax
from jax.experimental import pallas as pl
from jax.experimental.pallas import tpu as pltpu
from jax.experimental.pallas import tpu_sc as plsc

U = 50000
D = 64
E = 800000
BATCH = 4096
HALF = U // 2
ACC = 25088
PADGAP = ACC - HALF
UPAD = 2 * ACC
PAD_LOCAL = HALF
EPT = E // 16
CHUNK = 2048
NCH = EPT // CHUNK
TAIL = EPT - NCH * CHUNK
TAILF = (TAIL + 127) // 128
RPT = ACC // 16

_i32 = jnp.int32
_f32 = jnp.float32


def _iota16():
    return lax.iota(_i32, 16)


def _make_layer(in_rows, padgap_in):
    mesh = plsc.VectorSubcoreMesh(core_axis_name="c", subcore_axis_name="s")

    @functools.partial(
        pl.kernel,
        out_type=jax.ShapeDtypeStruct((UPAD, D), _f32),
        mesh=mesh,
        compiler_params=pltpu.CompilerParams(use_tc_tiling_on_sc=False, needs_layout_passes=False),
        scratch_types=[
            pltpu.VMEM((CHUNK,), _i32),
            pltpu.VMEM((CHUNK,), _i32),
            pltpu.VMEM((128,), _i32),
            pltpu.VMEM((128,), _i32),
            pltpu.VMEM((128,), _i32),
            pltpu.VMEM((128,), _i32),
            pltpu.VMEM((128, D), _f32),
            pltpu.VMEM((128, D), _f32),
            pltpu.VMEM((128,), _f32),
            pltpu.VMEM((128,), _f32),
            pltpu.VMEM_SHARED((ACC, D), _f32),
            pltpu.VMEM_SHARED((ACC,), _f32),
            pltpu.SemaphoreType.DMA,
            pltpu.SemaphoreType.DMA,
        ],
    )
    def layer(src_h, dst_h, h_h, ones_h, zcol_h, zrow_h, out_h,
              src_st, dst_st, g0, g1, s0, s1, rb0, rb1,
              ones_v, degb, acc, deg, sem0, sem1):
        c = lax.axis_index("c")
        t = lax.axis_index("s")
        row_base = c * HALF

        pltpu.sync_copy(ones_h, ones_v)
        pltpu.sync_copy(zcol_h, degb)
        pltpu.sync_copy(zrow_h, rb0)

        z0 = t * RPT

        def zero_body(s, _):
            pltpu.sync_copy(rb0, acc.at[pl.ds(z0 + s * 128, 128)])
            pltpu.sync_copy(degb, deg.at[pl.ds(z0 + s * 128, 128)])
            return _
        lax.fori_loop(0, 12, zero_body, None)
        pltpu.sync_copy(rb0.at[pl.ds(0, 32)], acc.at[pl.ds(z0 + 1536, 32)])
        pltpu.sync_copy(degb.at[pl.ds(0, 32)], deg.at[pl.ds(z0 + 1536, 32)])
        plsc.subcore_barrier()

        ebase = t * EPT

        def stage(e0, n):
            pltpu.sync_copy(src_h.at[pl.ds(e0, n)], src_st.at[pl.ds(0, n)])
            pltpu.sync_copy(dst_h.at[pl.ds(e0, n)], dst_st.at[pl.ds(0, n)])

        def build(off, s_ref, g_ref):
            for i in range(8):
                sl = pl.ds(off + i * 16, 16)
                d = dst_st[sl]
                s = src_st[sl]
                local = d - row_base
                m = (local >= 0) & (local < HALF)
                s_ref[pl.ds(i * 16, 16)] = jnp.where(m, local, PAD_LOCAL)
                if padgap_in:
                    s = s + jnp.where(s >= HALF, padgap_in, 0)
                g_ref[pl.ds(i * 16, 16)] = s

        def scatter(rb, s_ref):
            pltpu.sync_copy(rb, acc.at[s_ref], add=True)
            pltpu.sync_copy(ones_v, deg.at[s_ref], add=True)

        def chunk_body(ci, _):
            stage(ebase + ci * CHUNK, CHUNK)
            build(0, s0, g0)
            pltpu.async_copy(h_h.at[g0], rb0, sem0)

            def f2_body(f2, __):
                off1 = (2 * f2 + 1) * 128
                build(off1, s1, g1)
                d1 = pltpu.async_copy(h_h.at[g1], rb1, sem1)
                pltpu.make_async_copy(h_h.at[g0], rb0, sem0).wait()
                scatter(rb0, s0)

                @pl.when(f2 < 7)
                def _():
                    build((2 * f2 + 2) * 128, s0, g0)
                    pltpu.async_copy(h_h.at[g0], rb0, sem0)

                d1.wait()
                scatter(rb1, s1)
                return __
            lax.fori_loop(0, 8, f2_body, None)
            return _
        lax.fori_loop(0, NCH, chunk_body, None)

        stage(ebase + NCH * CHUNK, TAIL)
        for i in range(TAILF * 128 // 16 - TAIL // 16):
            sl = pl.ds(TAIL + i * 16, 16)
            dst_st[sl] = jnp.full((16,), -1, _i32)
            src_st[sl] = jnp.zeros((16,), _i32)
        for f in range(TAILF):
            build(f * 128, s0, g0)
            pltpu.async_copy(h_h.at[g0], rb0, sem0).wait()
            scatter(rb0, s0)
        plsc.subcore_barrier()

        def div_sub(r0, n):
            pltpu.sync_copy(acc.at[pl.ds(r0, n)], rb0.at[pl.ds(0, n)])
            pltpu.sync_copy(deg.at[pl.ds(r0, n)], degb.at[pl.ds(0, n)])

            def rg_body(rg, _):
                dv = degb[pl.ds(rg * 16, 16)]
                inv = 1.0 / jnp.maximum(dv, 1.0)
                for l in range(16):
                    sc = inv[l]
                    r = rg * 16 + l
                    for cc in range(4):
                        csl = pl.ds(cc * 16, 16)
                        rb0[r, csl] = rb0[r, csl] * sc
                return _
            if n == 128:
                lax.fori_loop(0, 8, rg_body, None)
            else:
                for rg in range(n // 16):
                    rg_body(rg, None)
            pltpu.sync_copy(rb0.at[pl.ds(0, n)],
                            out_h.at[pl.ds(c * ACC + r0, n)])

        def div_body(s, _):
            div_sub(z0 + s * 128, 128)
            return _
        lax.fori_loop(0, 12, div_body, None)
        div_sub(z0 + 1536, 32)

    return layer


_layer_first = _make_layer(U, 0)
_layer_next = _make_layer(UPAD, PADGAP)


_pred_mesh = plsc.VectorSubcoreMesh(core_axis_name="c", subcore_axis_name="s")


@functools.partial(
    pl.kernel,
    out_type=(jax.ShapeDtypeStruct((BATCH,), _f32),
              jax.ShapeDtypeStruct((BATCH, D), _f32),
              jax.ShapeDtypeStruct((BATCH, D), _f32)),
    mesh=_pred_mesh,
    compiler_params=pltpu.CompilerParams(use_tc_tiling_on_sc=False, needs_layout_passes=False),
    scratch_types=[
        pltpu.VMEM((128,), _i32),
        pltpu.VMEM((128,), _i32),
        pltpu.VMEM((128,), _i32),
        pltpu.VMEM((128, D), _f32),
        pltpu.VMEM((128, D), _f32),
        pltpu.VMEM((128, D), _f32),
        pltpu.VMEM((128, D), _f32),
        pltpu.VMEM((256,), _f32),
        pltpu.VMEM((128,), _f32),
        pltpu.SemaphoreType.DMA,
        pltpu.SemaphoreType.DMA,
        pltpu.SemaphoreType.DMA,
        pltpu.SemaphoreType.DMA,
    ],
)
def _predict(users_h, items_h, ue_h, h1_h, h2_h, ie_h,
             pred_h, lu_h, li_h,
             uix, iix, upx, bu, b1, b2, bi, pv, pred,
             semu, sem1, sem2, semi):
    c = lax.axis_index("c")
    t = lax.axis_index("s")
    wid = c * 16 + t
    base = wid * 128

    pltpu.sync_copy(users_h.at[pl.ds(base, 128)], uix)
    pltpu.sync_copy(items_h.at[pl.ds(base, 128)], iix)
    du = pltpu.async_copy(ue_h.at[uix], bu, semu)
    di = pltpu.async_copy(ie_h.at[iix], bi, semi)
    for i in range(8):
        sl = pl.ds(i * 16, 16)
        u = uix[sl]
        upx[sl] = u + jnp.where(u >= HALF, PADGAP, 0)
    d1 = pltpu.async_copy(h1_h.at[upx], b1, sem1)
    d2 = pltpu.async_copy(h2_h.at[upx], b2, sem2)
    du.wait()
    d1.wait()
    d2.wait()
    di.wait()

    def rg_body(rg, _):
        for l in range(16):
            r = rg * 16 + l
            acc_v = jnp.zeros((16,), _f32)
            for cc in range(4):
                csl = pl.ds(cc * 16, 16)
                u3 = bu[r, csl] + b1[r, csl] + b2[r, csl]
                bu[r, csl] = u3
                acc_v = acc_v + u3 * bi[r, csl]
            pv[pl.ds(l * 16, 16)] = acc_v
        dot = jnp.zeros((16,), _f32)
        for cc2 in range(16):
            dot = dot + plsc.load_gather(
                pv, [_iota16() * 16 + cc2])
        p = 1.0 / (1.0 + jnp.exp(-dot))
        pred[pl.ds(rg * 16, 16)] = p
        return _
    lax.fori_loop(0, 8, rg_body, None)

    pltpu.sync_copy(pred, pred_h.at[pl.ds(base, 128)])
    pltpu.sync_copy(bu, lu_h.at[pl.ds(base, 128)])
    pltpu.sync_copy(bi, li_h.at[pl.ds(base, 128)])


def kernel(users, items, edge_index, user_emb, item_emb):
    src = edge_index[0].astype(_i32)
    dst = edge_index[1].astype(_i32)
    users = users.astype(_i32)
    items = items.astype(_i32)
    ones = jnp.ones((128,), _f32)
    zcol = jnp.zeros((128,), _f32)
    zrow = jnp.zeros((128, D), _f32)
    h1 = _layer_first(src, dst, user_emb, ones, zcol, zrow)
    h2 = _layer_next(src, dst, h1, ones, zcol, zrow)
    return _predict(users, items, user_emb, h1, h2, item_emb)


# speedup vs baseline: 6.3099x; 1.0324x over previous
import functools

import jax
import jax.numpy as jnp
from jax import lax
from jax.experimental import pallas as pl
from jax.experimental.pallas import tpu as pltpu
from jax.experimental.pallas import tpu_sc as plsc

U = 50000
D = 64
E = 800000
BATCH = 4096
HALF = U // 2
ACC = 25088
PADGAP = ACC - HALF
UPAD = 2 * ACC
PAD_LOCAL = HALF
EPT = E // 16
CHUNK = 2048
NCH = EPT // CHUNK
TAIL = EPT - NCH * CHUNK
TAILF = (TAIL + 127) // 128
RPT = ACC // 16

_i32 = jnp.int32
_f32 = jnp.float32


def _iota16():
    return lax.iota(_i32, 16)


def _make_layer(in_rows, padgap_in):
    mesh = plsc.VectorSubcoreMesh(core_axis_name="c", subcore_axis_name="s")

    @functools.partial(
        pl.kernel,
        out_type=jax.ShapeDtypeStruct((UPAD, D), _f32),
        mesh=mesh,
        compiler_params=pltpu.CompilerParams(use_tc_tiling_on_sc=False, needs_layout_passes=False),
        scratch_types=[
            pltpu.VMEM((CHUNK,), _i32),
            pltpu.VMEM((CHUNK,), _i32),
            pltpu.VMEM((128,), _i32),
            pltpu.VMEM((128,), _i32),
            pltpu.VMEM((128,), _i32),
            pltpu.VMEM((128,), _i32),
            pltpu.VMEM((128, D), _f32),
            pltpu.VMEM((128, D), _f32),
            pltpu.VMEM((128,), _f32),
            pltpu.VMEM((128,), _f32),
            pltpu.VMEM_SHARED((ACC, D), _f32),
            pltpu.VMEM_SHARED((ACC,), _f32),
            pltpu.SemaphoreType.DMA,
            pltpu.SemaphoreType.DMA,
            pltpu.SemaphoreType.DMA,
        ],
    )
    def layer(edge_h, h_h, ones_h, zcol_h, zrow_h, out_h,
              src_st, dst_st, g0, g1, s0, s1, rb0, rb1,
              ones_v, degb, acc, deg, sem0, sem1, semA):
        c = lax.axis_index("c")
        t = lax.axis_index("s")
        row_base = c * HALF

        pltpu.sync_copy(ones_h, ones_v)
        pltpu.sync_copy(zcol_h, degb)
        pltpu.sync_copy(zrow_h, rb0)

        z0 = t * RPT

        def zero_body(s, _):
            pltpu.sync_copy(rb0, acc.at[pl.ds(z0 + s * 128, 128)])
            pltpu.sync_copy(degb, deg.at[pl.ds(z0 + s * 128, 128)])
            return _
        lax.fori_loop(0, 12, zero_body, None)
        pltpu.sync_copy(rb0.at[pl.ds(0, 32)], acc.at[pl.ds(z0 + 1536, 32)])
        pltpu.sync_copy(degb.at[pl.ds(0, 32)], deg.at[pl.ds(z0 + 1536, 32)])
        plsc.subcore_barrier()

        ebase = t * EPT

        def stage(e0, n):
            pltpu.sync_copy(edge_h.at[0, pl.ds(e0, n)], src_st.at[pl.ds(0, n)])
            pltpu.sync_copy(edge_h.at[1, pl.ds(e0, n)], dst_st.at[pl.ds(0, n)])

        def build(off, s_ref, g_ref):
            for i in range(8):
                sl = pl.ds(off + i * 16, 16)
                d = dst_st[sl]
                s = src_st[sl]
                local = d - row_base
                m = (local >= 0) & (local < HALF)
                s_ref[pl.ds(i * 16, 16)] = jnp.where(m, local, PAD_LOCAL)
                if padgap_in:
                    s = s + jnp.where(s >= HALF, padgap_in, 0)
                g_ref[pl.ds(i * 16, 16)] = s

        def scatter(rb, s_ref, semA):
            d = pltpu.async_copy(rb, acc.at[s_ref], semA, add=True)
            pltpu.sync_copy(ones_v, deg.at[s_ref], add=True)
            d.wait()

        def chunk_body(ci, _):
            stage(ebase + ci * CHUNK, CHUNK)
            build(0, s0, g0)
            pltpu.async_copy(h_h.at[g0], rb0, sem0)

            def f2_body(f2, __):
                off1 = (2 * f2 + 1) * 128
                build(off1, s1, g1)
                d1 = pltpu.async_copy(h_h.at[g1], rb1, sem1)
                pltpu.make_async_copy(h_h.at[g0], rb0, sem0).wait()
                scatter(rb0, s0, semA)

                @pl.when(f2 < 7)
                def _():
                    build((2 * f2 + 2) * 128, s0, g0)
                    pltpu.async_copy(h_h.at[g0], rb0, sem0)

                d1.wait()
                scatter(rb1, s1, semA)
                return __
            lax.fori_loop(0, 8, f2_body, None)
            return _
        lax.fori_loop(0, NCH, chunk_body, None)

        stage(ebase + NCH * CHUNK, TAIL)
        for i in range(TAILF * 128 // 16 - TAIL // 16):
            sl = pl.ds(TAIL + i * 16, 16)
            dst_st[sl] = jnp.full((16,), -1, _i32)
            src_st[sl] = jnp.zeros((16,), _i32)
        for f in range(TAILF):
            build(f * 128, s0, g0)
            pltpu.async_copy(h_h.at[g0], rb0, sem0).wait()
            scatter(rb0, s0, semA)
        plsc.subcore_barrier()

        def div_sub(r0, n):
            pltpu.sync_copy(acc.at[pl.ds(r0, n)], rb0.at[pl.ds(0, n)])
            pltpu.sync_copy(deg.at[pl.ds(r0, n)], degb.at[pl.ds(0, n)])

            def rg_body(rg, _):
                dv = degb[pl.ds(rg * 16, 16)]
                inv = 1.0 / jnp.maximum(dv, 1.0)
                for l in range(16):
                    sc = inv[l]
                    r = rg * 16 + l
                    for cc in range(4):
                        csl = pl.ds(cc * 16, 16)
                        rb0[r, csl] = rb0[r, csl] * sc
                return _
            if n == 128:
                lax.fori_loop(0, 8, rg_body, None)
            else:
                for rg in range(n // 16):
                    rg_body(rg, None)
            pltpu.sync_copy(rb0.at[pl.ds(0, n)],
                            out_h.at[pl.ds(c * ACC + r0, n)])

        def div_body(s, _):
            div_sub(z0 + s * 128, 128)
            return _
        lax.fori_loop(0, 12, div_body, None)
        div_sub(z0 + 1536, 32)

    return layer


_layer_first = _make_layer(U, 0)
_layer_next = _make_layer(UPAD, PADGAP)


_pred_mesh = plsc.VectorSubcoreMesh(core_axis_name="c", subcore_axis_name="s")


@functools.partial(
    pl.kernel,
    out_type=(jax.ShapeDtypeStruct((BATCH,), _f32),
              jax.ShapeDtypeStruct((BATCH, D), _f32),
              jax.ShapeDtypeStruct((BATCH, D), _f32)),
    mesh=_pred_mesh,
    compiler_params=pltpu.CompilerParams(use_tc_tiling_on_sc=False, needs_layout_passes=False),
    scratch_types=[
        pltpu.VMEM((128,), _i32),
        pltpu.VMEM((128,), _i32),
        pltpu.VMEM((128,), _i32),
        pltpu.VMEM((128, D), _f32),
        pltpu.VMEM((128, D), _f32),
        pltpu.VMEM((128, D), _f32),
        pltpu.VMEM((128, D), _f32),
        pltpu.VMEM((256,), _f32),
        pltpu.VMEM((128,), _f32),
        pltpu.SemaphoreType.DMA,
        pltpu.SemaphoreType.DMA,
        pltpu.SemaphoreType.DMA,
        pltpu.SemaphoreType.DMA,
    ],
)
def _predict(users_h, items_h, ue_h, h1_h, h2_h, ie_h,
             pred_h, lu_h, li_h,
             uix, iix, upx, bu, b1, b2, bi, pv, pred,
             semu, sem1, sem2, semi):
    c = lax.axis_index("c")
    t = lax.axis_index("s")
    wid = c * 16 + t
    base = wid * 128

    pltpu.sync_copy(users_h.at[pl.ds(base, 128)], uix)
    pltpu.sync_copy(items_h.at[pl.ds(base, 128)], iix)
    du = pltpu.async_copy(ue_h.at[uix], bu, semu)
    di = pltpu.async_copy(ie_h.at[iix], bi, semi)
    for i in range(8):
        sl = pl.ds(i * 16, 16)
        u = uix[sl]
        upx[sl] = u + jnp.where(u >= HALF, PADGAP, 0)
    d1 = pltpu.async_copy(h1_h.at[upx], b1, sem1)
    d2 = pltpu.async_copy(h2_h.at[upx], b2, sem2)
    du.wait()
    d1.wait()
    d2.wait()
    di.wait()

    def rg_body(rg, _):
        for l in range(16):
            r = rg * 16 + l
            acc_v = jnp.zeros((16,), _f32)
            for cc in range(4):
                csl = pl.ds(cc * 16, 16)
                u3 = bu[r, csl] + b1[r, csl] + b2[r, csl]
                bu[r, csl] = u3
                acc_v = acc_v + u3 * bi[r, csl]
            pv[pl.ds(l * 16, 16)] = acc_v
        dot = jnp.zeros((16,), _f32)
        for cc2 in range(16):
            dot = dot + plsc.load_gather(
                pv, [_iota16() * 16 + cc2])
        p = 1.0 / (1.0 + jnp.exp(-dot))
        pred[pl.ds(rg * 16, 16)] = p
        return _
    lax.fori_loop(0, 8, rg_body, None)

    pltpu.sync_copy(pred, pred_h.at[pl.ds(base, 128)])
    pltpu.sync_copy(bu, lu_h.at[pl.ds(base, 128)])
    pltpu.sync_copy(bi, li_h.at[pl.ds(base, 128)])


def kernel(users, items, edge_index, user_emb, item_emb):
    edge_index = edge_index.astype(_i32)
    users = users.astype(_i32)
    items = items.astype(_i32)
    ones = jnp.ones((128,), _f32)
    zcol = jnp.zeros((128,), _f32)
    zrow = jnp.zeros((128, D), _f32)
    h1 = _layer_first(edge_index, user_emb, ones, zcol, zrow)
    h2 = _layer_next(edge_index, h1, ones, zcol, zrow)
    return _predict(users, items, user_emb, h1, h2, item_emb)


# speedup vs baseline: 12.4205x; 1.9684x over previous
import functools

import jax
import jax.numpy as jnp
from jax import lax
from jax.experimental import pallas as pl
from jax.experimental.pallas import tpu as pltpu
from jax.experimental.pallas import tpu_sc as plsc

U = 50000
D = 64
E = 800000
BATCH = 4096
HALF = U // 2
ACC = 25088
PADGAP = ACC - HALF
UPAD = 2 * ACC
PAD_LOCAL = HALF
EPT = E // 16
CHUNK = 2048
NCH = EPT // CHUNK
TAIL = EPT - NCH * CHUNK
PCAP = CHUNK + 128
RPT = ACC // 16

_i32 = jnp.int32
_f32 = jnp.float32


def _iota16():
    return lax.iota(_i32, 16)


def _make_layer(in_rows, padgap_in):
    mesh = plsc.VectorSubcoreMesh(core_axis_name="c", subcore_axis_name="s")

    @functools.partial(
        pl.kernel,
        out_type=jax.ShapeDtypeStruct((UPAD, D), _f32),
        mesh=mesh,
        compiler_params=pltpu.CompilerParams(use_tc_tiling_on_sc=False, needs_layout_passes=False),
        scratch_types=[
            pltpu.VMEM((CHUNK,), _i32),
            pltpu.VMEM((CHUNK,), _i32),
            pltpu.VMEM((128,), _i32),
            pltpu.VMEM((128,), _i32),
            pltpu.VMEM((128,), _i32),
            pltpu.VMEM((128,), _i32),
            pltpu.VMEM((128, D), _f32),
            pltpu.VMEM((128, D), _f32),
            pltpu.VMEM((128,), _f32),
            pltpu.VMEM((128,), _f32),
            pltpu.VMEM((PCAP,), _i32),
            pltpu.VMEM((PCAP,), _i32),
            pltpu.VMEM_SHARED((ACC, D), _f32),
            pltpu.VMEM_SHARED((ACC,), _f32),
            pltpu.SemaphoreType.DMA,
            pltpu.SemaphoreType.DMA,
            pltpu.SemaphoreType.DMA,
        ],
    )
    def layer(edge_h, h_h, ones_h, zcol_h, zrow_h, out_h,
              src_st, dst_st, g0, g1, s0, s1, rb0, rb1,
              ones_v, degb, pend_src, pend_dst, acc, deg, sem0, sem1, semA):
        c = lax.axis_index("c")
        t = lax.axis_index("s")
        row_base = c * HALF

        pltpu.sync_copy(ones_h, ones_v)
        pltpu.sync_copy(zcol_h, degb)
        pltpu.sync_copy(zrow_h, rb0)

        z0 = t * RPT

        def zero_body(s, _):
            pltpu.sync_copy(rb0, acc.at[pl.ds(z0 + s * 128, 128)])
            pltpu.sync_copy(degb, deg.at[pl.ds(z0 + s * 128, 128)])
            return _
        lax.fori_loop(0, 12, zero_body, None)
        pltpu.sync_copy(rb0.at[pl.ds(0, 32)], acc.at[pl.ds(z0 + 1536, 32)])
        pltpu.sync_copy(degb.at[pl.ds(0, 32)], deg.at[pl.ds(z0 + 1536, 32)])
        plsc.subcore_barrier()

        ebase = t * EPT

        def stage(e0, n):
            pltpu.sync_copy(edge_h.at[0, pl.ds(e0, n)], src_st.at[pl.ds(0, n)])
            pltpu.sync_copy(edge_h.at[1, pl.ds(e0, n)], dst_st.at[pl.ds(0, n)])

        def compact(ngroups, cur):
            def g_body(g, cur):
                sl = pl.ds(g * 16, 16)
                d = dst_st[sl]
                s = src_st[sl]
                local = d - row_base
                m = (local >= 0) & (local < HALF)
                if padgap_in:
                    s = s + jnp.where(s >= HALF, padgap_in, 0)
                plsc.store_compressed(pend_src.at[pl.ds(cur, 16)], s, mask=m)
                plsc.store_compressed(pend_dst.at[pl.ds(cur, 16)], local,
                                      mask=m)
                cnt = plsc.all_reduce_population_count(m)[0]
                return cur + cnt
            return lax.fori_loop(0, ngroups, g_body, cur)

        def prep(fidx, s_ref, g_ref):
            for i in range(8):
                sl = pl.ds(i * 16, 16)
                psl = pl.ds(fidx * 128 + i * 16, 16)
                s_ref[sl] = pend_dst[psl]
                g_ref[sl] = pend_src[psl]

        def scatter(rb, s_ref, semA):
            d = pltpu.async_copy(rb, acc.at[s_ref], semA, add=True)
            pltpu.sync_copy(ones_v, deg.at[s_ref], add=True)
            d.wait()

        def drain(cur):
            nfire = cur // 128

            @pl.when(nfire > 0)
            def _():
                prep(0, s0, g0)
                pltpu.async_copy(h_h.at[g0], rb0, sem0)

            def f2_body(f2, __):
                f_a = 2 * f2
                f_b = f_a + 1

                @pl.when(f_b < nfire)
                def _():
                    prep(f_b, s1, g1)
                    pltpu.async_copy(h_h.at[g1], rb1, sem1)

                @pl.when(f_a < nfire)
                def _():
                    pltpu.make_async_copy(h_h.at[g0], rb0, sem0).wait()
                    scatter(rb0, s0, semA)

                @pl.when(f_a + 2 < nfire)
                def _():
                    prep(f_a + 2, s0, g0)
                    pltpu.async_copy(h_h.at[g0], rb0, sem0)

                @pl.when(f_b < nfire)
                def _():
                    pltpu.make_async_copy(h_h.at[g1], rb1, sem1).wait()
                    scatter(rb1, s1, semA)
                return __
            lax.fori_loop(0, (nfire + 1) // 2, f2_body, None)
            for i in range(8):
                sl = pl.ds(i * 16, 16)
                psl = pl.ds(nfire * 128 + i * 16, 16)
                pend_src[sl] = pend_src[psl]
                pend_dst[sl] = pend_dst[psl]
            return cur - nfire * 128

        def chunk_body(ci, cur):
            stage(ebase + ci * CHUNK, CHUNK)
            cur = compact(CHUNK // 16, cur)
            return drain(cur)
        cur = lax.fori_loop(0, NCH, chunk_body, 0)
        stage(ebase + NCH * CHUNK, TAIL)
        cur = compact(TAIL // 16, cur)
        cur = drain(cur)
        b0 = (cur // 16) * 16
        for i in range(8):
            sl = pl.ds(b0 + i * 16, 16)
            pos = b0 + i * 16 + _iota16()
            keep = pos < cur
            pend_dst[sl] = jnp.where(keep, pend_dst[sl], PAD_LOCAL)
            pend_src[sl] = jnp.where(keep, pend_src[sl], 0)
        prep(0, s0, g0)
        pltpu.async_copy(h_h.at[g0], rb0, sem0).wait()
        scatter(rb0, s0, semA)
        plsc.subcore_barrier()

        def div_sub(r0, n):
            pltpu.sync_copy(acc.at[pl.ds(r0, n)], rb0.at[pl.ds(0, n)])
            pltpu.sync_copy(deg.at[pl.ds(r0, n)], degb.at[pl.ds(0, n)])

            def rg_body(rg, _):
                dv = degb[pl.ds(rg * 16, 16)]
                inv = 1.0 / jnp.maximum(dv, 1.0)
                for l in range(16):
                    sc = inv[l]
                    r = rg * 16 + l
                    for cc in range(4):
                        csl = pl.ds(cc * 16, 16)
                        rb0[r, csl] = rb0[r, csl] * sc
                return _
            if n == 128:
                lax.fori_loop(0, 8, rg_body, None)
            else:
                for rg in range(n // 16):
                    rg_body(rg, None)
            pltpu.sync_copy(rb0.at[pl.ds(0, n)],
                            out_h.at[pl.ds(c * ACC + r0, n)])

        def div_body(s, _):
            div_sub(z0 + s * 128, 128)
            return _
        lax.fori_loop(0, 12, div_body, None)
        div_sub(z0 + 1536, 32)

    return layer


_layer_first = _make_layer(U, 0)
_layer_next = _make_layer(UPAD, PADGAP)


_pred_mesh = plsc.VectorSubcoreMesh(core_axis_name="c", subcore_axis_name="s")


@functools.partial(
    pl.kernel,
    out_type=(jax.ShapeDtypeStruct((BATCH,), _f32),
              jax.ShapeDtypeStruct((BATCH, D), _f32),
              jax.ShapeDtypeStruct((BATCH, D), _f32)),
    mesh=_pred_mesh,
    compiler_params=pltpu.CompilerParams(use_tc_tiling_on_sc=False, needs_layout_passes=False),
    scratch_types=[
        pltpu.VMEM((128,), _i32),
        pltpu.VMEM((128,), _i32),
        pltpu.VMEM((128,), _i32),
        pltpu.VMEM((128, D), _f32),
        pltpu.VMEM((128, D), _f32),
        pltpu.VMEM((128, D), _f32),
        pltpu.VMEM((128, D), _f32),
        pltpu.VMEM((256,), _f32),
        pltpu.VMEM((128,), _f32),
        pltpu.SemaphoreType.DMA,
        pltpu.SemaphoreType.DMA,
        pltpu.SemaphoreType.DMA,
        pltpu.SemaphoreType.DMA,
    ],
)
def _predict(users_h, items_h, ue_h, h1_h, h2_h, ie_h,
             pred_h, lu_h, li_h,
             uix, iix, upx, bu, b1, b2, bi, pv, pred,
             semu, sem1, sem2, semi):
    c = lax.axis_index("c")
    t = lax.axis_index("s")
    wid = c * 16 + t
    base = wid * 128

    pltpu.sync_copy(users_h.at[pl.ds(base, 128)], uix)
    pltpu.sync_copy(items_h.at[pl.ds(base, 128)], iix)
    du = pltpu.async_copy(ue_h.at[uix], bu, semu)
    di = pltpu.async_copy(ie_h.at[iix], bi, semi)
    for i in range(8):
        sl = pl.ds(i * 16, 16)
        u = uix[sl]
        upx[sl] = u + jnp.where(u >= HALF, PADGAP, 0)
    d1 = pltpu.async_copy(h1_h.at[upx], b1, sem1)
    d2 = pltpu.async_copy(h2_h.at[upx], b2, sem2)
    du.wait()
    d1.wait()
    d2.wait()
    di.wait()

    def rg_body(rg, _):
        for l in range(16):
            r = rg * 16 + l
            acc_v = jnp.zeros((16,), _f32)
            for cc in range(4):
                csl = pl.ds(cc * 16, 16)
                u3 = bu[r, csl] + b1[r, csl] + b2[r, csl]
                bu[r, csl] = u3
                acc_v = acc_v + u3 * bi[r, csl]
            pv[pl.ds(l * 16, 16)] = acc_v
        dot = jnp.zeros((16,), _f32)
        for cc2 in range(16):
            dot = dot + plsc.load_gather(
                pv, [_iota16() * 16 + cc2])
        p = 1.0 / (1.0 + jnp.exp(-dot))
        pred[pl.ds(rg * 16, 16)] = p
        return _
    lax.fori_loop(0, 8, rg_body, None)

    pltpu.sync_copy(pred, pred_h.at[pl.ds(base, 128)])
    pltpu.sync_copy(bu, lu_h.at[pl.ds(base, 128)])
    pltpu.sync_copy(bi, li_h.at[pl.ds(base, 128)])


def kernel(users, items, edge_index, user_emb, item_emb):
    edge_index = edge_index.astype(_i32)
    users = users.astype(_i32)
    items = items.astype(_i32)
    ones = jnp.ones((128,), _f32)
    zcol = jnp.zeros((128,), _f32)
    zrow = jnp.zeros((128, D), _f32)
    h1 = _layer_first(edge_index, user_emb, ones, zcol, zrow)
    h2 = _layer_next(edge_index, h1, ones, zcol, zrow)
    return _predict(users, items, user_emb, h1, h2, item_emb)


# speedup vs baseline: 12.4316x; 1.0009x over previous
import functools

import jax
import jax.numpy as jnp
from jax import lax
from jax.experimental import pallas as pl
from jax.experimental.pallas import tpu as pltpu
from jax.experimental.pallas import tpu_sc as plsc

U = 50000
D = 64
E = 800000
BATCH = 4096
HALF = U // 2
ACC = 25088
PADGAP = ACC - HALF
UPAD = 2 * ACC
PAD_LOCAL = HALF
EPT = E // 16
CHUNK = 2048
NCH = EPT // CHUNK
TAIL = EPT - NCH * CHUNK
PCAP = CHUNK + 128
RPT = ACC // 16

_i32 = jnp.int32
_f32 = jnp.float32


def _iota16():
    return lax.iota(_i32, 16)


def _make_layer(in_rows, padgap_in):
    mesh = plsc.VectorSubcoreMesh(core_axis_name="c", subcore_axis_name="s")

    @functools.partial(
        pl.kernel,
        out_type=jax.ShapeDtypeStruct((UPAD, D), _f32),
        mesh=mesh,
        compiler_params=pltpu.CompilerParams(use_tc_tiling_on_sc=False, needs_layout_passes=False),
        scratch_types=[
            pltpu.VMEM((CHUNK,), _i32),
            pltpu.VMEM((CHUNK,), _i32),
            pltpu.VMEM((128,), _i32),
            pltpu.VMEM((128,), _i32),
            pltpu.VMEM((128,), _i32),
            pltpu.VMEM((128,), _i32),
            pltpu.VMEM((128, D), _f32),
            pltpu.VMEM((128, D), _f32),
            pltpu.VMEM((128,), _f32),
            pltpu.VMEM((128,), _f32),
            pltpu.VMEM((PCAP,), _i32),
            pltpu.VMEM((PCAP,), _i32),
            pltpu.VMEM_SHARED((ACC, D), _f32),
            pltpu.VMEM_SHARED((ACC,), _f32),
            pltpu.SemaphoreType.DMA,
            pltpu.SemaphoreType.DMA,
            pltpu.SemaphoreType.DMA,
        ],
    )
    def layer(edge_h, h_h, ones_h, zcol_h, zrow_h, out_h,
              src_st, dst_st, g0, g1, s0, s1, rb0, rb1,
              ones_v, degb, pend_src, pend_dst, acc, deg, sem0, sem1, semA):
        c = lax.axis_index("c")
        t = lax.axis_index("s")
        row_base = c * HALF

        pltpu.sync_copy(ones_h, ones_v)
        pltpu.sync_copy(zcol_h, degb)
        pltpu.sync_copy(zrow_h, rb0)

        z0 = t * RPT

        def zero_body(s, _):
            pltpu.sync_copy(rb0, acc.at[pl.ds(z0 + s * 128, 128)])
            pltpu.sync_copy(degb, deg.at[pl.ds(z0 + s * 128, 128)])
            return _
        lax.fori_loop(0, 12, zero_body, None)
        pltpu.sync_copy(rb0.at[pl.ds(0, 32)], acc.at[pl.ds(z0 + 1536, 32)])
        pltpu.sync_copy(degb.at[pl.ds(0, 32)], deg.at[pl.ds(z0 + 1536, 32)])
        plsc.subcore_barrier()

        ebase = t * EPT

        def stage(e0, n):
            pltpu.sync_copy(edge_h.at[0, pl.ds(e0, n)], src_st.at[pl.ds(0, n)])
            pltpu.sync_copy(edge_h.at[1, pl.ds(e0, n)], dst_st.at[pl.ds(0, n)])

        def compact(ngroups, cur):
            def g_body(g, cur):
                sl = pl.ds(g * 16, 16)
                d = dst_st[sl]
                s = src_st[sl]
                local = d - row_base
                m = (local >= 0) & (local < HALF)
                if padgap_in:
                    s = s + jnp.where(s >= HALF, padgap_in, 0)
                plsc.store_compressed(pend_src.at[pl.ds(cur, 16)], s, mask=m)
                plsc.store_compressed(pend_dst.at[pl.ds(cur, 16)], local,
                                      mask=m)
                cnt = plsc.all_reduce_population_count(m)[0]
                return cur + cnt
            return lax.fori_loop(0, ngroups, g_body, cur)

        def prep(fidx, s_ref, g_ref):
            for i in range(8):
                sl = pl.ds(i * 16, 16)
                psl = pl.ds(fidx * 128 + i * 16, 16)
                s_ref[sl] = pend_dst[psl]
                g_ref[sl] = pend_src[psl]

        def scatter(rb, s_ref, semA):
            d = pltpu.async_copy(rb, acc.at[s_ref], semA, add=True)
            pltpu.sync_copy(ones_v, deg.at[s_ref], add=True)
            d.wait()

        def drain(cur):
            nfire = cur // 128

            @pl.when(nfire > 0)
            def _():
                prep(0, s0, g0)
                pltpu.async_copy(h_h.at[g0], rb0, sem0)

            def f2_body(f2, __):
                f_a = 2 * f2
                f_b = f_a + 1

                @pl.when(f_b < nfire)
                def _():
                    prep(f_b, s1, g1)
                    pltpu.async_copy(h_h.at[g1], rb1, sem1)

                @pl.when(f_a < nfire)
                def _():
                    pltpu.make_async_copy(h_h.at[g0], rb0, sem0).wait()
                    scatter(rb0, s0, semA)

                @pl.when(f_a + 2 < nfire)
                def _():
                    prep(f_a + 2, s0, g0)
                    pltpu.async_copy(h_h.at[g0], rb0, sem0)

                @pl.when(f_b < nfire)
                def _():
                    pltpu.make_async_copy(h_h.at[g1], rb1, sem1).wait()
                    scatter(rb1, s1, semA)
                return __
            lax.fori_loop(0, (nfire + 1) // 2, f2_body, None)
            for i in range(8):
                sl = pl.ds(i * 16, 16)
                psl = pl.ds(nfire * 128 + i * 16, 16)
                pend_src[sl] = pend_src[psl]
                pend_dst[sl] = pend_dst[psl]
            return cur - nfire * 128

        def chunk_body(ci, cur):
            stage(ebase + ci * CHUNK, CHUNK)
            cur = compact(CHUNK // 16, cur)
            return drain(cur)
        with jax.named_scope("edge_phase"):
            cur = lax.fori_loop(0, NCH, chunk_body, 0)
        stage(ebase + NCH * CHUNK, TAIL)
        cur = compact(TAIL // 16, cur)
        cur = drain(cur)
        b0 = (cur // 16) * 16
        for i in range(8):
            sl = pl.ds(b0 + i * 16, 16)
            pos = b0 + i * 16 + _iota16()
            keep = pos < cur
            pend_dst[sl] = jnp.where(keep, pend_dst[sl], PAD_LOCAL)
            pend_src[sl] = jnp.where(keep, pend_src[sl], 0)
        prep(0, s0, g0)
        pltpu.async_copy(h_h.at[g0], rb0, sem0).wait()
        scatter(rb0, s0, semA)
        plsc.subcore_barrier()

        def div_sub(r0, n):
            pltpu.sync_copy(acc.at[pl.ds(r0, n)], rb0.at[pl.ds(0, n)])
            pltpu.sync_copy(deg.at[pl.ds(r0, n)], degb.at[pl.ds(0, n)])

            def rg_body(rg, _):
                dv = degb[pl.ds(rg * 16, 16)]
                inv = 1.0 / jnp.maximum(dv, 1.0)
                for l in range(16):
                    sc = inv[l]
                    r = rg * 16 + l
                    for cc in range(4):
                        csl = pl.ds(cc * 16, 16)
                        rb0[r, csl] = rb0[r, csl] * sc
                return _
            if n == 128:
                lax.fori_loop(0, 8, rg_body, None)
            else:
                for rg in range(n // 16):
                    rg_body(rg, None)
            pltpu.sync_copy(rb0.at[pl.ds(0, n)],
                            out_h.at[pl.ds(c * ACC + r0, n)])

        def div_body(s, _):
            div_sub(z0 + s * 128, 128)
            return _
        with jax.named_scope("div_phase"):
            lax.fori_loop(0, 12, div_body, None)
            div_sub(z0 + 1536, 32)

    return layer


_layer_first = _make_layer(U, 0)
_layer_next = _make_layer(UPAD, PADGAP)


_pred_mesh = plsc.VectorSubcoreMesh(core_axis_name="c", subcore_axis_name="s")


@functools.partial(
    pl.kernel,
    out_type=(jax.ShapeDtypeStruct((BATCH,), _f32),
              jax.ShapeDtypeStruct((BATCH, D), _f32),
              jax.ShapeDtypeStruct((BATCH, D), _f32)),
    mesh=_pred_mesh,
    compiler_params=pltpu.CompilerParams(use_tc_tiling_on_sc=False, needs_layout_passes=False),
    scratch_types=[
        pltpu.VMEM((128,), _i32),
        pltpu.VMEM((128,), _i32),
        pltpu.VMEM((128,), _i32),
        pltpu.VMEM((128, D), _f32),
        pltpu.VMEM((128, D), _f32),
        pltpu.VMEM((128, D), _f32),
        pltpu.VMEM((128, D), _f32),
        pltpu.VMEM((256,), _f32),
        pltpu.VMEM((128,), _f32),
        pltpu.SemaphoreType.DMA,
        pltpu.SemaphoreType.DMA,
        pltpu.SemaphoreType.DMA,
        pltpu.SemaphoreType.DMA,
    ],
)
def _predict(users_h, items_h, ue_h, h1_h, h2_h, ie_h,
             pred_h, lu_h, li_h,
             uix, iix, upx, bu, b1, b2, bi, pv, pred,
             semu, sem1, sem2, semi):
    c = lax.axis_index("c")
    t = lax.axis_index("s")
    wid = c * 16 + t
    base = wid * 128

    pltpu.sync_copy(users_h.at[pl.ds(base, 128)], uix)
    pltpu.sync_copy(items_h.at[pl.ds(base, 128)], iix)
    du = pltpu.async_copy(ue_h.at[uix], bu, semu)
    di = pltpu.async_copy(ie_h.at[iix], bi, semi)
    for i in range(8):
        sl = pl.ds(i * 16, 16)
        u = uix[sl]
        upx[sl] = u + jnp.where(u >= HALF, PADGAP, 0)
    d1 = pltpu.async_copy(h1_h.at[upx], b1, sem1)
    d2 = pltpu.async_copy(h2_h.at[upx], b2, sem2)
    du.wait()
    d1.wait()
    d2.wait()
    di.wait()

    def rg_body(rg, _):
        for l in range(16):
            r = rg * 16 + l
            acc_v = jnp.zeros((16,), _f32)
            for cc in range(4):
                csl = pl.ds(cc * 16, 16)
                u3 = bu[r, csl] + b1[r, csl] + b2[r, csl]
                bu[r, csl] = u3
                acc_v = acc_v + u3 * bi[r, csl]
            pv[pl.ds(l * 16, 16)] = acc_v
        dot = jnp.zeros((16,), _f32)
        for cc2 in range(16):
            dot = dot + plsc.load_gather(
                pv, [_iota16() * 16 + cc2])
        p = 1.0 / (1.0 + jnp.exp(-dot))
        pred[pl.ds(rg * 16, 16)] = p
        return _
    lax.fori_loop(0, 8, rg_body, None)

    pltpu.sync_copy(pred, pred_h.at[pl.ds(base, 128)])
    pltpu.sync_copy(bu, lu_h.at[pl.ds(base, 128)])
    pltpu.sync_copy(bi, li_h.at[pl.ds(base, 128)])


def kernel(users, items, edge_index, user_emb, item_emb):
    edge_index = edge_index.astype(_i32)
    users = users.astype(_i32)
    items = items.astype(_i32)
    ones = jnp.ones((128,), _f32)
    zcol = jnp.zeros((128,), _f32)
    zrow = jnp.zeros((128, D), _f32)
    h1 = _layer_first(edge_index, user_emb, ones, zcol, zrow)
    h2 = _layer_next(edge_index, h1, ones, zcol, zrow)
    return _predict(users, items, user_emb, h1, h2, item_emb)


# speedup vs baseline: 13.3731x; 1.0757x over previous
import functools

import jax
import jax.numpy as jnp
from jax import lax
from jax.experimental import pallas as pl
from jax.experimental.pallas import tpu as pltpu
from jax.experimental.pallas import tpu_sc as plsc

U = 50000
D = 64
E = 800000
BATCH = 4096
HALF = U // 2
ACC = 25088
PADGAP = ACC - HALF
UPAD = 2 * ACC
PAD_LOCAL = HALF
EPT = E // 16
CHUNK = 2048
NCH = EPT // CHUNK
TAIL = EPT - NCH * CHUNK
PCAP = CHUNK + 128
RPT = ACC // 16

_i32 = jnp.int32
_f32 = jnp.float32


def _iota16():
    return lax.iota(_i32, 16)


def _make_layer(in_rows, padgap_in):
    mesh = plsc.VectorSubcoreMesh(core_axis_name="c", subcore_axis_name="s")

    @functools.partial(
        pl.kernel,
        out_type=jax.ShapeDtypeStruct((UPAD, D), _f32),
        mesh=mesh,
        compiler_params=pltpu.CompilerParams(use_tc_tiling_on_sc=False, needs_layout_passes=False),
        scratch_types=[
            pltpu.VMEM((CHUNK,), _i32),
            pltpu.VMEM((CHUNK,), _i32),
            pltpu.VMEM((128,), _i32),
            pltpu.VMEM((128,), _i32),
            pltpu.VMEM((128,), _i32),
            pltpu.VMEM((128,), _i32),
            pltpu.VMEM((128, D), _f32),
            pltpu.VMEM((128, D), _f32),
            pltpu.VMEM((128,), _f32),
            pltpu.VMEM((128,), _f32),
            pltpu.VMEM((PCAP,), _i32),
            pltpu.VMEM((PCAP,), _i32),
            pltpu.VMEM_SHARED((ACC, D), _f32),
            pltpu.VMEM_SHARED((ACC,), _f32),
            pltpu.SemaphoreType.DMA,
            pltpu.SemaphoreType.DMA,
            pltpu.SemaphoreType.DMA,
        ],
    )
    def layer(edge_h, h_h, ones_h, zcol_h, zrow_h, out_h,
              src_st, dst_st, g0, g1, s0, s1, rb0, rb1,
              ones_v, degb, pend_src, pend_dst, acc, deg, sem0, sem1, semA):
        c = lax.axis_index("c")
        t = lax.axis_index("s")
        row_base = c * HALF

        pltpu.sync_copy(ones_h, ones_v)
        pltpu.sync_copy(zcol_h, degb)
        pltpu.sync_copy(zrow_h, rb0)

        z0 = t * RPT

        def zero_body(s, _):
            pltpu.sync_copy(rb0, acc.at[pl.ds(z0 + s * 128, 128)])
            pltpu.sync_copy(degb, deg.at[pl.ds(z0 + s * 128, 128)])
            return _
        lax.fori_loop(0, 12, zero_body, None)
        pltpu.sync_copy(rb0.at[pl.ds(0, 32)], acc.at[pl.ds(z0 + 1536, 32)])
        pltpu.sync_copy(degb.at[pl.ds(0, 32)], deg.at[pl.ds(z0 + 1536, 32)])
        plsc.subcore_barrier()

        ebase = t * EPT

        def stage(e0, n):
            pltpu.sync_copy(edge_h.at[0, pl.ds(e0, n)], src_st.at[pl.ds(0, n)])
            pltpu.sync_copy(edge_h.at[1, pl.ds(e0, n)], dst_st.at[pl.ds(0, n)])

        def _scan_groups(base_group, n, cur):
            ms, ss, ls = [], [], []
            for i in range(n):
                sl = pl.ds((base_group + i) * 16, 16)
                d = dst_st[sl]
                s = src_st[sl]
                local = d - row_base
                m = (local >= 0) & (local < HALF)
                if padgap_in:
                    s = s + jnp.where(s >= HALF, padgap_in, 0)
                ms.append(m)
                ss.append(s)
                ls.append(local)
            cnts = [plsc.all_reduce_population_count(m)[0] for m in ms]
            for i in range(n):
                plsc.store_compressed(pend_src.at[pl.ds(cur, 16)], ss[i],
                                      mask=ms[i])
                plsc.store_compressed(pend_dst.at[pl.ds(cur, 16)], ls[i],
                                      mask=ms[i])
                cur = cur + cnts[i]
            return cur

        def compact(ngroups, cur):
            nblocks = ngroups // 8

            def b_body(b, cur):
                return _scan_groups(b * 8, 8, cur)
            cur = lax.fori_loop(0, nblocks, b_body, cur)
            if ngroups % 8:
                cur = _scan_groups(nblocks * 8, ngroups % 8, cur)
            return cur

        def prep(fidx, s_ref, g_ref):
            for i in range(8):
                sl = pl.ds(i * 16, 16)
                psl = pl.ds(fidx * 128 + i * 16, 16)
                s_ref[sl] = pend_dst[psl]
                g_ref[sl] = pend_src[psl]

        def scatter(rb, s_ref, semA):
            d = pltpu.async_copy(rb, acc.at[s_ref], semA, add=True)
            pltpu.sync_copy(ones_v, deg.at[s_ref], add=True)
            d.wait()

        def drain(cur):
            nfire = cur // 128

            @pl.when(nfire > 0)
            def _():
                prep(0, s0, g0)
                pltpu.async_copy(h_h.at[g0], rb0, sem0)

            def f2_body(f2, __):
                f_a = 2 * f2
                f_b = f_a + 1

                @pl.when(f_b < nfire)
                def _():
                    prep(f_b, s1, g1)
                    pltpu.async_copy(h_h.at[g1], rb1, sem1)

                @pl.when(f_a < nfire)
                def _():
                    pltpu.make_async_copy(h_h.at[g0], rb0, sem0).wait()
                    scatter(rb0, s0, semA)

                @pl.when(f_a + 2 < nfire)
                def _():
                    prep(f_a + 2, s0, g0)
                    pltpu.async_copy(h_h.at[g0], rb0, sem0)

                @pl.when(f_b < nfire)
                def _():
                    pltpu.make_async_copy(h_h.at[g1], rb1, sem1).wait()
                    scatter(rb1, s1, semA)
                return __
            lax.fori_loop(0, (nfire + 1) // 2, f2_body, None)
            for i in range(8):
                sl = pl.ds(i * 16, 16)
                psl = pl.ds(nfire * 128 + i * 16, 16)
                pend_src[sl] = pend_src[psl]
                pend_dst[sl] = pend_dst[psl]
            return cur - nfire * 128

        def chunk_body(ci, cur):
            stage(ebase + ci * CHUNK, CHUNK)
            cur = compact(CHUNK // 16, cur)
            return drain(cur)
        with jax.named_scope("edge_phase"):
            cur = lax.fori_loop(0, NCH, chunk_body, 0)
        stage(ebase + NCH * CHUNK, TAIL)
        cur = compact(TAIL // 16, cur)
        cur = drain(cur)
        b0 = (cur // 16) * 16
        for i in range(8):
            sl = pl.ds(b0 + i * 16, 16)
            pos = b0 + i * 16 + _iota16()
            keep = pos < cur
            pend_dst[sl] = jnp.where(keep, pend_dst[sl], PAD_LOCAL)
            pend_src[sl] = jnp.where(keep, pend_src[sl], 0)
        prep(0, s0, g0)
        pltpu.async_copy(h_h.at[g0], rb0, sem0).wait()
        scatter(rb0, s0, semA)
        plsc.subcore_barrier()

        def div_sub(r0, n):
            pltpu.sync_copy(acc.at[pl.ds(r0, n)], rb0.at[pl.ds(0, n)])
            pltpu.sync_copy(deg.at[pl.ds(r0, n)], degb.at[pl.ds(0, n)])

            def rg_body(rg, _):
                dv = degb[pl.ds(rg * 16, 16)]
                inv = 1.0 / jnp.maximum(dv, 1.0)
                for l in range(16):
                    sc = inv[l]
                    r = rg * 16 + l
                    for cc in range(4):
                        csl = pl.ds(cc * 16, 16)
                        rb0[r, csl] = rb0[r, csl] * sc
                return _
            if n == 128:
                lax.fori_loop(0, 8, rg_body, None)
            else:
                for rg in range(n // 16):
                    rg_body(rg, None)
            pltpu.sync_copy(rb0.at[pl.ds(0, n)],
                            out_h.at[pl.ds(c * ACC + r0, n)])

        def div_body(s, _):
            div_sub(z0 + s * 128, 128)
            return _
        with jax.named_scope("div_phase"):
            lax.fori_loop(0, 12, div_body, None)
            div_sub(z0 + 1536, 32)

    return layer


_layer_first = _make_layer(U, 0)
_layer_next = _make_layer(UPAD, PADGAP)


_pred_mesh = plsc.VectorSubcoreMesh(core_axis_name="c", subcore_axis_name="s")


@functools.partial(
    pl.kernel,
    out_type=(jax.ShapeDtypeStruct((BATCH,), _f32),
              jax.ShapeDtypeStruct((BATCH, D), _f32),
              jax.ShapeDtypeStruct((BATCH, D), _f32)),
    mesh=_pred_mesh,
    compiler_params=pltpu.CompilerParams(use_tc_tiling_on_sc=False, needs_layout_passes=False),
    scratch_types=[
        pltpu.VMEM((128,), _i32),
        pltpu.VMEM((128,), _i32),
        pltpu.VMEM((128,), _i32),
        pltpu.VMEM((128, D), _f32),
        pltpu.VMEM((128, D), _f32),
        pltpu.VMEM((128, D), _f32),
        pltpu.VMEM((128, D), _f32),
        pltpu.VMEM((256,), _f32),
        pltpu.VMEM((128,), _f32),
        pltpu.SemaphoreType.DMA,
        pltpu.SemaphoreType.DMA,
        pltpu.SemaphoreType.DMA,
        pltpu.SemaphoreType.DMA,
    ],
)
def _predict(users_h, items_h, ue_h, h1_h, h2_h, ie_h,
             pred_h, lu_h, li_h,
             uix, iix, upx, bu, b1, b2, bi, pv, pred,
             semu, sem1, sem2, semi):
    c = lax.axis_index("c")
    t = lax.axis_index("s")
    wid = c * 16 + t
    base = wid * 128

    pltpu.sync_copy(users_h.at[pl.ds(base, 128)], uix)
    pltpu.sync_copy(items_h.at[pl.ds(base, 128)], iix)
    du = pltpu.async_copy(ue_h.at[uix], bu, semu)
    di = pltpu.async_copy(ie_h.at[iix], bi, semi)
    for i in range(8):
        sl = pl.ds(i * 16, 16)
        u = uix[sl]
        upx[sl] = u + jnp.where(u >= HALF, PADGAP, 0)
    d1 = pltpu.async_copy(h1_h.at[upx], b1, sem1)
    d2 = pltpu.async_copy(h2_h.at[upx], b2, sem2)
    du.wait()
    d1.wait()
    d2.wait()
    di.wait()

    def rg_body(rg, _):
        for l in range(16):
            r = rg * 16 + l
            acc_v = jnp.zeros((16,), _f32)
            for cc in range(4):
                csl = pl.ds(cc * 16, 16)
                u3 = bu[r, csl] + b1[r, csl] + b2[r, csl]
                bu[r, csl] = u3
                acc_v = acc_v + u3 * bi[r, csl]
            pv[pl.ds(l * 16, 16)] = acc_v
        dot = jnp.zeros((16,), _f32)
        for cc2 in range(16):
            dot = dot + plsc.load_gather(
                pv, [_iota16() * 16 + cc2])
        p = 1.0 / (1.0 + jnp.exp(-dot))
        pred[pl.ds(rg * 16, 16)] = p
        return _
    lax.fori_loop(0, 8, rg_body, None)

    pltpu.sync_copy(pred, pred_h.at[pl.ds(base, 128)])
    pltpu.sync_copy(bu, lu_h.at[pl.ds(base, 128)])
    pltpu.sync_copy(bi, li_h.at[pl.ds(base, 128)])


def kernel(users, items, edge_index, user_emb, item_emb):
    edge_index = edge_index.astype(_i32)
    users = users.astype(_i32)
    items = items.astype(_i32)
    ones = jnp.ones((128,), _f32)
    zcol = jnp.zeros((128,), _f32)
    zrow = jnp.zeros((128, D), _f32)
    h1 = _layer_first(edge_index, user_emb, ones, zcol, zrow)
    h2 = _layer_next(edge_index, h1, ones, zcol, zrow)
    return _predict(users, items, user_emb, h1, h2, item_emb)
